# Initial kernel scaffold; baseline (speedup 1.0000x reference)
import functools

import jax
import jax.numpy as jnp
from jax import lax
from jax.experimental import pallas as pl
from jax.experimental.pallas import tpu as pltpu
from jax.experimental.pallas import tpu_sc as plsc

f32 = jnp.float32
i32 = jnp.int32

N = 10000
E = 160000
SN = 5000
SE = 80000
B = 256
D = 300
L = 5

WH = 160
DP = 304
HID = 608

EP = 163840
SEP = 81920
NPAD = 10016
SNPAD = 5008

_HI = lax.Precision.HIGHEST


def _dot(a, b):
    return jnp.dot(a, b, precision=_HI, preferred_element_type=f32)


def _dot0(a, b):
    return lax.dot_general(a, b, (((0,), (0,)), ((), ())), precision=_HI,
                           preferred_element_type=f32)


def _make_gather_scatter(nn, npad, nch):
    rows_pt = npad // 16
    mesh = plsc.VectorSubcoreMesh(core_axis_name="c", subcore_axis_name="s")

    @functools.partial(
        pl.kernel, mesh=mesh,
        out_type=jax.ShapeDtypeStruct((2, npad, WH), f32),
        scratch_types=[
            pltpu.VMEM((nch, 128), i32),
            pltpu.VMEM((nch, 128), i32),
            pltpu.VMEM((128, WH), f32),
            pltpu.VMEM_SHARED((npad, WH), f32),
            pltpu.SemaphoreType.DMA,
        ])
    def k(h_hbm, srcs_hbm, dsts_hbm, zeros_hbm, out_hbm, sidx, didx, rows, agg, sem):
        c = lax.axis_index("c")
        s = lax.axis_index("s")
        w = c * 16 + s
        pltpu.sync_copy(zeros_hbm.at[pl.ds(0, rows_pt)],
                        agg.at[pl.ds(s * rows_pt, rows_pt)])
        pltpu.sync_copy(srcs_hbm.at[w], sidx)
        pltpu.sync_copy(dsts_hbm.at[s], didx)
        plsc.subcore_barrier()

        def body(j, carry):
            pltpu.async_copy(h_hbm.at[sidx.at[j]], rows, sem).wait()
            pltpu.sync_copy(rows, agg.at[didx.at[j]], add=True)
            return carry

        lax.fori_loop(0, nch, body, 0)
        plsc.subcore_barrier()
        pltpu.sync_copy(agg.at[pl.ds(s * rows_pt, rows_pt)],
                        out_hbm.at[c, pl.ds(s * rows_pt, rows_pt)])

    return k


def _make_count_scatter(npad, nch2):
    rows_pt = npad // 16
    mesh = plsc.VectorSubcoreMesh(core_axis_name="c", subcore_axis_name="s")

    @functools.partial(
        pl.kernel, mesh=mesh,
        out_type=jax.ShapeDtypeStruct((2, npad, 16), f32),
        scratch_types=[
            pltpu.VMEM((nch2, 128), i32),
            pltpu.VMEM((128, 16), f32),
            pltpu.VMEM_SHARED((npad, 16), f32),
            pltpu.SemaphoreType.DMA,
        ])
    def k(oh_hbm, dsts_hbm, zeros_hbm, out_hbm, didx, rows, cnt, sem):
        c = lax.axis_index("c")
        s = lax.axis_index("s")
        w = c * 16 + s
        pltpu.sync_copy(zeros_hbm.at[pl.ds(0, rows_pt)],
                        cnt.at[pl.ds(s * rows_pt, rows_pt)])
        pltpu.sync_copy(dsts_hbm.at[w], didx)
        plsc.subcore_barrier()

        def body(j, carry):
            pltpu.sync_copy(oh_hbm.at[w, pl.ds(j * 128, 128)], rows)
            pltpu.sync_copy(rows, cnt.at[didx.at[j]], add=True)
            return carry

        lax.fori_loop(0, nch2, body, 0)
        plsc.subcore_barrier()
        pltpu.sync_copy(cnt.at[pl.ds(s * rows_pt, rows_pt)],
                        out_hbm.at[c, pl.ds(s * rows_pt, rows_pt)])

    return k


def _make_onehot(ep, rb):
    nblk = ep // rb

    def body(attr_ref, out_ref):
        def blk(i, carry):
            r0 = pl.multiple_of(i * rb, 8)
            ab = attr_ref[pl.ds(r0, rb), :]
            code = ab[:, 0:1] * 3 + ab[:, 1:2]
            out_ref[pl.ds(r0, rb), :] = (
                code == lax.broadcasted_iota(i32, (rb, 16), 1)).astype(f32)
            return carry
        lax.fori_loop(0, nblk, blk, 0)

    return pl.pallas_call(
        body, out_shape=jax.ShapeDtypeStruct((ep, 16), f32))


def _make_embed(nn, rb):
    nblk = nn // rb

    def body(x_ref, e1a, e1b, e2a, e2b, out_ref):
        def blk(i, carry):
            r0 = pl.multiple_of(i * rb, 8)
            xb = x_ref[pl.ds(r0, rb), :]
            oh1 = (xb[:, 0:1] == lax.broadcasted_iota(i32, (rb, 120), 1)).astype(f32)
            oh2 = (xb[:, 1:2] == lax.broadcasted_iota(i32, (rb, 8), 1)).astype(f32)
            out_ref[0, pl.ds(r0, rb), :] = _dot(oh1, e1a[...]) + _dot(oh2, e2a[...])
            out_ref[1, pl.ds(r0, rb), :] = _dot(oh1, e1b[...]) + _dot(oh2, e2b[...])
            return carry
        lax.fori_loop(0, nblk, blk, 0)

    return pl.pallas_call(
        body, out_shape=jax.ShapeDtypeStruct((2, nn, WH), f32))


def _make_mlp(nn, npad, rb, relu_out):
    nblk = nn // rb
    inv_n = 1.0 / nn

    def body(hh, agg, cc, ta, tb, sa, sb, w1a, w1b, b1, w2a, w2b, b2a, b2b,
             ga, gb, ba, bb, out, h2a_s, h2b_s, stats):
        stats[...] = jnp.zeros((8, WH), f32)

        def blk(i, carry):
            r0 = pl.multiple_of(i * rb, 8)
            ha = hh[0, pl.ds(r0, rb), :]
            hb = hh[1, pl.ds(r0, rb), :]
            aa = agg[0, pl.ds(r0, rb), :]
            ab = agg[1, pl.ds(r0, rb), :]
            cb = cc[0, pl.ds(r0, rb), :] + cc[1, pl.ds(r0, rb), :]
            za = aa + ha + sa[...] + _dot(cb, ta[...])
            zb = ab + hb + sb[...] + _dot(cb, tb[...])
            hid = jnp.maximum(_dot(za, w1a[...]) + _dot(zb, w1b[...]) + b1[...], 0.0)
            h2a = _dot(hid, w2a[...]) + b2a[...]
            h2b = _dot(hid, w2b[...]) + b2b[...]
            h2a_s[pl.ds(r0, rb), :] = h2a
            h2b_s[pl.ds(r0, rb), :] = h2b
            stats[0:1, :] += jnp.sum(h2a, 0, keepdims=True)
            stats[1:2, :] += jnp.sum(h2a * h2a, 0, keepdims=True)
            stats[2:3, :] += jnp.sum(h2b, 0, keepdims=True)
            stats[3:4, :] += jnp.sum(h2b * h2b, 0, keepdims=True)
            return carry

        lax.fori_loop(0, nblk, blk, 0)

        mua = stats[0:1, :] * inv_n
        mub = stats[2:3, :] * inv_n
        siga = ga[...] * lax.rsqrt(stats[1:2, :] * inv_n - mua * mua + 1e-5)
        sigb = gb[...] * lax.rsqrt(stats[3:4, :] * inv_n - mub * mub + 1e-5)
        ca = ba[...] - mua * siga
        cb2 = bb[...] - mub * sigb

        def blk2(i, carry):
            r0 = pl.multiple_of(i * rb, 8)
            ha = h2a_s[pl.ds(r0, rb), :] * siga + ca
            hb = h2b_s[pl.ds(r0, rb), :] * sigb + cb2
            if relu_out:
                ha = jnp.maximum(ha, 0.0)
                hb = jnp.maximum(hb, 0.0)
            out[0, pl.ds(r0, rb), :] = ha
            out[1, pl.ds(r0, rb), :] = hb
            return carry

        lax.fori_loop(0, nblk, blk2, 0)

    return pl.pallas_call(
        body,
        out_shape=jax.ShapeDtypeStruct((2, nn, WH), f32),
        scratch_shapes=[
            pltpu.VMEM((nn, WH), f32),
            pltpu.VMEM((nn, WH), f32),
            pltpu.VMEM((8, WH), f32),
        ])


def _make_pool(nn, rb, threshold):
    nblk = nn // rb

    def body(hh, p1a, p1b, pb1, p2, pb2, seg8, m8, s_out, c_out):
        s_out[...] = jnp.zeros((B, DP), f32)
        c_out[...] = jnp.zeros((B, 8), f32)

        def blk(i, carry):
            r0 = pl.multiple_of(i * rb, 8)
            ha = hh[0, pl.ds(r0, rb), :]
            hb = hh[1, pl.ds(r0, rb), :]
            hid = jnp.maximum(_dot(ha, p1a[...]) + _dot(hb, p1b[...]) + pb1[...], 0.0)
            nf = _dot(hid, p2[...]) + pb2[...]
            wcol = m8[pl.ds(r0, rb), 0:1]
            if threshold:
                wcol = (wcol > 0.5).astype(f32)
            segb = seg8[pl.ds(r0, rb), 0:1]
            oht = (segb == lax.broadcasted_iota(i32, (rb, B), 1)).astype(f32)
            s_out[...] += _dot0(oht, nf * wcol)
            c_out[...] += _dot0(oht, jnp.broadcast_to(wcol, (rb, 8)))
            return carry

        lax.fori_loop(0, nblk, blk, 0)

    return pl.pallas_call(
        body,
        out_shape=[jax.ShapeDtypeStruct((B, DP), f32),
                   jax.ShapeDtypeStruct((B, 8), f32)])


def _final_logits(s_g, c_g, s_s, c_s):
    def body(sg, cg, ss, cs, out):
        gf = sg[...] / jnp.maximum(cg[:, 0:1], 1.0)
        sf = ss[...] / jnp.maximum(cs[:, 0:1], 1.0)
        gn = gf / jnp.maximum(jnp.sqrt(jnp.sum(gf * gf, 1, keepdims=True)), 1e-12)
        sn = sf / jnp.maximum(jnp.sqrt(jnp.sum(sf * sf, 1, keepdims=True)), 1e-12)
        out[...] = lax.dot_general(gn, sn, (((1,), (1,)), ((), ())),
                                   precision=_HI, preferred_element_type=f32) * 10.0

    return pl.pallas_call(
        body, out_shape=jax.ShapeDtypeStruct((B, B), f32))(s_g, c_g, s_s, c_s)


def _split_cols(m):
    pad = [(0, 0)] * (m.ndim - 1)
    mp = jnp.pad(m, pad + [(0, DP - m.shape[-1])])
    return mp[..., :WH], jnp.pad(mp[..., WH:], pad + [(0, 16)])


def _split_rows(m):
    mp = jnp.pad(m, [(0, DP - m.shape[0]), (0, 0)])
    return mp[:WH], jnp.pad(mp[WH:], [(0, 16), (0, 0)])


def _prep_edges(edge_index, n_nodes, e_pad):
    e0 = edge_index.shape[1]
    src = edge_index[0]
    dst = edge_index[1]
    srcp = jnp.concatenate([src, jnp.zeros((e_pad - e0,), i32)])
    dstp = jnp.concatenate([dst, jnp.full((e_pad - e0,), n_nodes, i32)])
    src_off = jnp.stack([srcp, srcp + n_nodes])
    nch = e_pad // (16 * 128)
    srcs = src_off.reshape(32, nch, 128)
    dsts16 = dstp.reshape(16, nch, 128)
    dsts32 = dstp.reshape(32, (e_pad // 32))
    return srcs, dsts16, dsts32


def kernel(x, edge_index, edge_attr, scaffold_mask, batch, graph_contrast_labels,
           s_x, s_edge_index, s_edge_attr, s_batch, x_emb1, x_emb2, edge_emb1,
           edge_emb2, W1, b1, W2, b2, gamma, beta, P1, pb1, P2, pb2):
    e1a, e1b = _split_cols(x_emb1)
    e2a, e2b = _split_cols(jnp.pad(x_emb2, ((0, 5), (0, 0))))
    ks = jnp.arange(9)
    T = edge_emb1[:, ks // 3, :] + edge_emb2[:, ks % 3, :]
    T = jnp.pad(T, ((0, 0), (0, 7), (0, 0)))
    Ta, Tb = _split_cols(T)
    selfe = edge_emb1[:, 4, :] + edge_emb2[:, 0, :]
    sa, sb = _split_cols(selfe[:, None, :])
    W1p = jnp.pad(W1, ((0, 0), (0, DP - D), (0, HID - 2 * D)))
    W1a = W1p[:, :WH, :]
    W1b = jnp.pad(W1p[:, WH:, :], ((0, 0), (0, 16), (0, 0)))
    b1p = jnp.pad(b1, ((0, 0), (0, HID - 2 * D)))[:, None, :]
    W2p = jnp.pad(W2, ((0, 0), (0, HID - 2 * D), (0, 0)))
    W2a, W2b = _split_cols(W2p)
    b2a, b2b = _split_cols(b2[:, None, :])
    gaa, gab = _split_cols(gamma[:, None, :])
    baa, bab = _split_cols(beta[:, None, :])
    P1a, P1b = _split_rows(jnp.pad(P1, ((0, 0), (0, DP - D))))
    pb1p = jnp.pad(pb1, (0, DP - D))[None, :]
    P2p = jnp.pad(P2, ((0, DP - D), (0, DP - D)))
    pb2p = jnp.pad(pb2, (0, DP - D))[None, :]

    srcs_g, d16_g, d32_g = _prep_edges(edge_index, N, EP)
    srcs_s, d16_s, d32_s = _prep_edges(s_edge_index, SN, SEP)
    attr_g = jnp.pad(edge_attr, ((0, EP - E), (0, 6)))
    attr_s = jnp.pad(s_edge_attr, ((0, SEP - SE), (0, 6)))
    xg = jnp.pad(x, ((0, 0), (0, 6)))
    xs = jnp.pad(s_x, ((0, 0), (0, 6)))
    seg_g = jnp.pad(batch[:, None], ((0, 0), (0, 7)))
    seg_s = jnp.pad(s_batch[:, None], ((0, 0), (0, 7)))
    m_g = jnp.pad(scaffold_mask[:, None], ((0, 0), (0, 7)))
    m_s = jnp.ones((SN, 8), f32)
    zeros_wh = jnp.zeros((NPAD // 16, WH), f32)
    zeros_16 = jnp.zeros((NPAD // 16, 16), f32)

    gs_g = _make_gather_scatter(N, NPAD, EP // (16 * 128))
    gs_s = _make_gather_scatter(SN, SNPAD, SEP // (16 * 128))
    cnt_g = _make_count_scatter(NPAD, EP // (32 * 128))
    cnt_s = _make_count_scatter(SNPAD, SEP // (32 * 128))
    oh_g = _make_onehot(EP, 4096)
    oh_s = _make_onehot(SEP, 4096)
    emb_g = _make_embed(N, 400)
    emb_s = _make_embed(SN, 200)
    pool_g = _make_pool(N, 400, True)
    pool_s = _make_pool(SN, 200, False)

    hh_g = emb_g(xg, e1a, e1b, e2a, e2b)
    hh_s = emb_s(xs, e1a, e1b, e2a, e2b)
    ohg = oh_g(attr_g).reshape(32, EP // 32, 16)
    ohs = oh_s(attr_s).reshape(32, SEP // 32, 16)
    d32_g = d32_g.reshape(32, EP // 32)
    d32_s = d32_s.reshape(32, SEP // 32)
    C_g = cnt_g(ohg, d32_g, zeros_16)
    C_s = cnt_s(ohs, d32_s, zeros_16[:SNPAD // 16])

    for l in range(L):
        mlp_g = _make_mlp(N, NPAD, 400, l != L - 1)
        mlp_s = _make_mlp(SN, SNPAD, 200, l != L - 1)
        agg_g = gs_g(hh_g.reshape(2 * N, WH), srcs_g, d16_g, zeros_wh)
        agg_s = gs_s(hh_s.reshape(2 * SN, WH), srcs_s, d16_s, zeros_wh[:SNPAD // 16])
        hh_g = mlp_g(hh_g, agg_g, C_g, Ta[l], Tb[l], sa[l], sb[l], W1a[l], W1b[l],
                     b1p[l], W2a[l], W2b[l], b2a[l], b2b[l], gaa[l], gab[l],
                     baa[l], bab[l])
        hh_s = mlp_s(hh_s, agg_s, C_s, Ta[l], Tb[l], sa[l], sb[l], W1a[l], W1b[l],
                     b1p[l], W2a[l], W2b[l], b2a[l], b2b[l], gaa[l], gab[l],
                     baa[l], bab[l])

    S_g, c_g = pool_g(hh_g, P1a, P1b, pb1p, P2p, pb2p, seg_g, m_g)
    S_s, c_s = pool_s(hh_s, P1a, P1b, pb1p, P2p, pb2p, seg_s, m_s)
    logits = _final_logits(S_g, c_g, S_s, c_s)
    return (logits, graph_contrast_labels)


# speedup vs baseline: 3.0714x; 3.0714x over previous
import functools

import jax
import jax.numpy as jnp
from jax import lax
from jax.experimental import pallas as pl
from jax.experimental.pallas import tpu as pltpu
from jax.experimental.pallas import tpu_sc as plsc

f32 = jnp.float32
i32 = jnp.int32

N = 10000
E = 160000
SN = 5000
SE = 80000
B = 256
D = 300
L = 5

WH = 160
DP = 304
HID = 608

EP = 163840
SEP = 81920
NPAD = 10112
SNPAD = 5120

_HI = lax.Precision.HIGHEST


def _dot(a, b):
    return jnp.dot(a, b, precision=_HI, preferred_element_type=f32)


def _dot0(a, b):
    return lax.dot_general(a, b, (((0,), (0,)), ((), ())), precision=_HI,
                           preferred_element_type=f32)


def _make_gather_scatter(nn, npad, nch):
    rows_pt = npad // 16
    mesh = plsc.VectorSubcoreMesh(core_axis_name="c", subcore_axis_name="s")

    @functools.partial(
        pl.kernel, mesh=mesh,
        out_type=jax.ShapeDtypeStruct((2, npad, WH), f32),
        compiler_params=pltpu.CompilerParams(use_tc_tiling_on_sc=False),
        scratch_types=[
            pltpu.VMEM((128,), i32),
            pltpu.VMEM((128,), i32),
            pltpu.VMEM((128, WH), f32),
            pltpu.VMEM_SHARED((npad, WH), f32),
            pltpu.SemaphoreType.DMA,
        ])
    def k(h_hbm, srcs_hbm, dsts_hbm, zeros_hbm, out_hbm, sidx, didx, rows, agg, sem):
        c = lax.axis_index("c")
        s = lax.axis_index("s")
        w = c * 16 + s
        pltpu.sync_copy(zeros_hbm.at[pl.ds(0, rows_pt)],
                        agg.at[pl.ds(s * rows_pt, rows_pt)])
        plsc.subcore_barrier()

        def body(j, carry):
            pltpu.sync_copy(srcs_hbm.at[w, j], sidx)
            pltpu.sync_copy(dsts_hbm.at[s, j], didx)
            pltpu.async_copy(h_hbm.at[sidx], rows, sem).wait()
            pltpu.sync_copy(rows, agg.at[didx], add=True)
            return carry

        lax.fori_loop(0, nch, body, 0)
        plsc.subcore_barrier()
        pltpu.sync_copy(agg.at[pl.ds(s * rows_pt, rows_pt)],
                        out_hbm.at[c, pl.ds(s * rows_pt, rows_pt)])

    return k


def _make_count_scatter(npad, nch2):
    rows_pt = npad // 16
    mesh = plsc.VectorSubcoreMesh(core_axis_name="c", subcore_axis_name="s")

    @functools.partial(
        pl.kernel, mesh=mesh,
        out_type=jax.ShapeDtypeStruct((2, npad, 16), f32),
        compiler_params=pltpu.CompilerParams(use_tc_tiling_on_sc=False),
        scratch_types=[
            pltpu.VMEM((nch2, 128), i32),
            pltpu.VMEM((128, 16), f32),
            pltpu.VMEM_SHARED((npad, 16), f32),
            pltpu.SemaphoreType.DMA,
        ])
    def k(oh_hbm, dsts_hbm, zeros_hbm, out_hbm, didx, rows, cnt, sem):
        c = lax.axis_index("c")
        s = lax.axis_index("s")
        w = c * 16 + s
        pltpu.sync_copy(zeros_hbm.at[pl.ds(0, rows_pt)],
                        cnt.at[pl.ds(s * rows_pt, rows_pt)])
        pltpu.sync_copy(dsts_hbm.at[w], didx)
        plsc.subcore_barrier()

        def body(j, carry):
            pltpu.sync_copy(oh_hbm.at[w, pl.ds(j * 128, 128)], rows)
            pltpu.sync_copy(rows, cnt.at[didx.at[j]], add=True)
            return carry

        lax.fori_loop(0, nch2, body, 0)
        plsc.subcore_barrier()
        pltpu.sync_copy(cnt.at[pl.ds(s * rows_pt, rows_pt)],
                        out_hbm.at[c, pl.ds(s * rows_pt, rows_pt)])

    return k


def _whole(shape):
    return pl.BlockSpec(shape, lambda *_: tuple(0 for _ in shape))


def _make_onehot(ep, rb):
    def body(attr_ref, out_ref):
        ab = attr_ref[...]
        code = ab[:, 0:1] * 3 + ab[:, 1:2]
        out_ref[...] = (code == lax.broadcasted_iota(i32, (rb, 16), 1)).astype(f32)

    return pl.pallas_call(
        body,
        grid=(ep // rb,),
        in_specs=[pl.BlockSpec((rb, 8), lambda i: (i, 0))],
        out_specs=pl.BlockSpec((rb, 16), lambda i: (i, 0)),
        out_shape=jax.ShapeDtypeStruct((ep, 16), f32))


def _make_embed(nn, rb):
    def body(x_ref, e1a, e1b, e2a, e2b, out_ref):
        xb = x_ref[...]
        oh1 = (xb[:, 0:1] == lax.broadcasted_iota(i32, (rb, 120), 1)).astype(f32)
        oh2 = (xb[:, 1:2] == lax.broadcasted_iota(i32, (rb, 8), 1)).astype(f32)
        out_ref[0] = _dot(oh1, e1a[...]) + _dot(oh2, e2a[...])
        out_ref[1] = _dot(oh1, e1b[...]) + _dot(oh2, e2b[...])

    return pl.pallas_call(
        body,
        grid=(nn // rb,),
        in_specs=[pl.BlockSpec((rb, 8), lambda i: (i, 0)),
                  _whole((120, WH)), _whole((120, WH)),
                  _whole((8, WH)), _whole((8, WH))],
        out_specs=pl.BlockSpec((2, rb, WH), lambda i: (0, i, 0)),
        out_shape=jax.ShapeDtypeStruct((2, nn, WH), f32))


def _make_mlp(nn, rb, relu_out):
    nblk = nn // rb
    inv_n = 1.0 / nn

    def body(hh, agg, cc, ta, tb, sa, sb, w1a, w1b, b1, w2a, w2b, b2a, b2b,
             ga, gb, ba, bb, out, h2a_s, h2b_s, stats):
        p = pl.program_id(0)
        i = pl.program_id(1)
        r0 = pl.multiple_of(i * rb, 8)

        @pl.when(jnp.logical_and(p == 0, i == 0))
        def _():
            stats[...] = jnp.zeros((8, WH), f32)

        @pl.when(p == 0)
        def _():
            za = agg[0] + hh[0] + sa[...] + _dot(cc[0] + cc[1], ta[...])
            zb = agg[1] + hh[1] + sb[...] + _dot(cc[0] + cc[1], tb[...])
            hid = jnp.maximum(_dot(za, w1a[...]) + _dot(zb, w1b[...]) + b1[...],
                              0.0)
            h2a = _dot(hid, w2a[...]) + b2a[...]
            h2b = _dot(hid, w2b[...]) + b2b[...]
            h2a_s[pl.ds(r0, rb), :] = h2a
            h2b_s[pl.ds(r0, rb), :] = h2b
            stats[0:1, :] += jnp.sum(h2a, 0, keepdims=True)
            stats[1:2, :] += jnp.sum(h2a * h2a, 0, keepdims=True)
            stats[2:3, :] += jnp.sum(h2b, 0, keepdims=True)
            stats[3:4, :] += jnp.sum(h2b * h2b, 0, keepdims=True)

        @pl.when(p == 1)
        def _():
            mua = stats[0:1, :] * inv_n
            mub = stats[2:3, :] * inv_n
            siga = ga[...] * lax.rsqrt(stats[1:2, :] * inv_n - mua * mua + 1e-5)
            sigb = gb[...] * lax.rsqrt(stats[3:4, :] * inv_n - mub * mub + 1e-5)
            ha = h2a_s[pl.ds(r0, rb), :] * siga + (ba[...] - mua * siga)
            hb = h2b_s[pl.ds(r0, rb), :] * sigb + (bb[...] - mub * sigb)
            if relu_out:
                ha = jnp.maximum(ha, 0.0)
                hb = jnp.maximum(hb, 0.0)
            out[0] = ha
            out[1] = hb

    blk3 = lambda: pl.BlockSpec((2, rb, WH), lambda p, i: (0, i, 0))
    return pl.pallas_call(
        body,
        grid=(2, nblk),
        in_specs=[blk3(), blk3(),
                  pl.BlockSpec((2, rb, 16), lambda p, i: (0, i, 0)),
                  _whole((16, WH)), _whole((16, WH)),
                  _whole((1, WH)), _whole((1, WH)),
                  _whole((WH, HID)), _whole((WH, HID)), _whole((1, HID)),
                  _whole((HID, WH)), _whole((HID, WH)),
                  _whole((1, WH)), _whole((1, WH)),
                  _whole((1, WH)), _whole((1, WH)),
                  _whole((1, WH)), _whole((1, WH))],
        out_specs=blk3(),
        out_shape=jax.ShapeDtypeStruct((2, nn, WH), f32),
        scratch_shapes=[
            pltpu.VMEM((nn, WH), f32),
            pltpu.VMEM((nn, WH), f32),
            pltpu.VMEM((8, WH), f32),
        ])


def _make_pool(nn, rb, threshold):
    def body(hh, p1a, p1b, pb1, p2, pb2, seg8, m8, s_out, c_out):
        i = pl.program_id(0)

        @pl.when(i == 0)
        def _():
            s_out[...] = jnp.zeros((B, DP), f32)
            c_out[...] = jnp.zeros((B, 8), f32)

        hid = jnp.maximum(_dot(hh[0], p1a[...]) + _dot(hh[1], p1b[...]) + pb1[...],
                          0.0)
        nf = _dot(hid, p2[...]) + pb2[...]
        wcol = m8[:, 0:1]
        if threshold:
            wcol = (wcol > 0.5).astype(f32)
        segb = seg8[:, 0:1]
        oht = (segb == lax.broadcasted_iota(i32, (rb, B), 1)).astype(f32)
        s_out[...] += _dot0(oht, nf * wcol)
        c_out[...] += _dot0(oht, jnp.broadcast_to(wcol, (rb, 8)))

    return pl.pallas_call(
        body,
        grid=(nn // rb,),
        in_specs=[pl.BlockSpec((2, rb, WH), lambda i: (0, i, 0)),
                  _whole((WH, DP)), _whole((WH, DP)), _whole((1, DP)),
                  _whole((DP, DP)), _whole((1, DP)),
                  pl.BlockSpec((rb, 8), lambda i: (i, 0)),
                  pl.BlockSpec((rb, 8), lambda i: (i, 0))],
        out_specs=[pl.BlockSpec((B, DP), lambda i: (0, 0)),
                   pl.BlockSpec((B, 8), lambda i: (0, 0))],
        out_shape=[jax.ShapeDtypeStruct((B, DP), f32),
                   jax.ShapeDtypeStruct((B, 8), f32)])


def _final_logits(s_g, c_g, s_s, c_s):
    def body(sg, cg, ss, cs, out):
        gf = sg[...] / jnp.maximum(cg[:, 0:1], 1.0)
        sf = ss[...] / jnp.maximum(cs[:, 0:1], 1.0)
        gn = gf / jnp.maximum(jnp.sqrt(jnp.sum(gf * gf, 1, keepdims=True)), 1e-12)
        sn = sf / jnp.maximum(jnp.sqrt(jnp.sum(sf * sf, 1, keepdims=True)), 1e-12)
        out[...] = lax.dot_general(gn, sn, (((1,), (1,)), ((), ())),
                                   precision=_HI, preferred_element_type=f32) * 10.0

    return pl.pallas_call(
        body, out_shape=jax.ShapeDtypeStruct((B, B), f32))(s_g, c_g, s_s, c_s)


def _split_cols(m):
    pad = [(0, 0)] * (m.ndim - 1)
    mp = jnp.pad(m, pad + [(0, DP - m.shape[-1])])
    return mp[..., :WH], jnp.pad(mp[..., WH:], pad + [(0, 16)])


def _split_rows(m):
    mp = jnp.pad(m, [(0, DP - m.shape[0]), (0, 0)])
    return mp[:WH], jnp.pad(mp[WH:], [(0, 16), (0, 0)])


def _prep_edges(edge_index, n_nodes, e_pad):
    e0 = edge_index.shape[1]
    src = edge_index[0]
    dst = edge_index[1]
    srcp = jnp.concatenate([src, jnp.zeros((e_pad - e0,), i32)])
    dstp = jnp.concatenate([dst, jnp.full((e_pad - e0,), n_nodes, i32)])
    src_off = jnp.stack([srcp, srcp + n_nodes])
    nch = e_pad // (16 * 128)
    srcs = src_off.reshape(32, nch, 128)
    dsts16 = dstp.reshape(16, nch, 128)
    dsts32 = dstp.reshape(32, e_pad // (32 * 128), 128)
    return srcs, dsts16, dsts32


def kernel(x, edge_index, edge_attr, scaffold_mask, batch, graph_contrast_labels,
           s_x, s_edge_index, s_edge_attr, s_batch, x_emb1, x_emb2, edge_emb1,
           edge_emb2, W1, b1, W2, b2, gamma, beta, P1, pb1, P2, pb2):
    e1a, e1b = _split_cols(x_emb1)
    e2a, e2b = _split_cols(jnp.pad(x_emb2, ((0, 5), (0, 0))))
    ks = jnp.arange(9)
    T = edge_emb1[:, ks // 3, :] + edge_emb2[:, ks % 3, :]
    T = jnp.pad(T, ((0, 0), (0, 7), (0, 0)))
    Ta, Tb = _split_cols(T)
    selfe = edge_emb1[:, 4, :] + edge_emb2[:, 0, :]
    sa, sb = _split_cols(selfe[:, None, :])
    W1p = jnp.pad(W1, ((0, 0), (0, DP - D), (0, HID - 2 * D)))
    W1a = W1p[:, :WH, :]
    W1b = jnp.pad(W1p[:, WH:, :], ((0, 0), (0, 16), (0, 0)))
    b1p = jnp.pad(b1, ((0, 0), (0, HID - 2 * D)))[:, None, :]
    W2p = jnp.pad(W2, ((0, 0), (0, HID - 2 * D), (0, 0)))
    W2a, W2b = _split_cols(W2p)
    b2a, b2b = _split_cols(b2[:, None, :])
    gaa, gab = _split_cols(gamma[:, None, :])
    baa, bab = _split_cols(beta[:, None, :])
    P1a, P1b = _split_rows(jnp.pad(P1, ((0, 0), (0, DP - D))))
    pb1p = jnp.pad(pb1, (0, DP - D))[None, :]
    P2p = jnp.pad(P2, ((0, DP - D), (0, DP - D)))
    pb2p = jnp.pad(pb2, (0, DP - D))[None, :]

    srcs_g, d16_g, d32_g = _prep_edges(edge_index, N, EP)
    srcs_s, d16_s, d32_s = _prep_edges(s_edge_index, SN, SEP)
    attr_g = jnp.pad(edge_attr, ((0, EP - E), (0, 6)))
    attr_s = jnp.pad(s_edge_attr, ((0, SEP - SE), (0, 6)))
    xg = jnp.pad(x, ((0, 0), (0, 6)))
    xs = jnp.pad(s_x, ((0, 0), (0, 6)))
    seg_g = jnp.pad(batch[:, None], ((0, 0), (0, 7)))
    seg_s = jnp.pad(s_batch[:, None], ((0, 0), (0, 7)))
    m_g = jnp.pad(scaffold_mask[:, None], ((0, 0), (0, 7)))
    m_s = jnp.ones((SN, 8), f32)
    zeros_wh = jnp.zeros((NPAD // 16, WH), f32)
    zeros_16 = jnp.zeros((NPAD // 16, 16), f32)

    gs_g = _make_gather_scatter(N, NPAD, EP // (16 * 128))
    gs_s = _make_gather_scatter(SN, SNPAD, SEP // (16 * 128))
    cnt_g = _make_count_scatter(NPAD, EP // (32 * 128))
    cnt_s = _make_count_scatter(SNPAD, SEP // (32 * 128))
    oh_g = _make_onehot(EP, 4096)
    oh_s = _make_onehot(SEP, 4096)
    emb_g = _make_embed(N, 400)
    emb_s = _make_embed(SN, 200)
    pool_g = _make_pool(N, 400, True)
    pool_s = _make_pool(SN, 200, False)

    hh_g = emb_g(xg, e1a, e1b, e2a, e2b)
    hh_s = emb_s(xs, e1a, e1b, e2a, e2b)
    ohg = oh_g(attr_g).reshape(32, EP // 32, 16)
    ohs = oh_s(attr_s).reshape(32, SEP // 32, 16)
    C_g = cnt_g(ohg, d32_g, zeros_16)
    C_s = cnt_s(ohs, d32_s, zeros_16[:SNPAD // 16])

    for l in range(L):
        mlp_g = _make_mlp(N, 400, l != L - 1)
        mlp_s = _make_mlp(SN, 200, l != L - 1)
        agg_g = gs_g(hh_g.reshape(2 * N, WH), srcs_g, d16_g, zeros_wh)
        agg_s = gs_s(hh_s.reshape(2 * SN, WH), srcs_s, d16_s, zeros_wh[:SNPAD // 16])
        hh_g = mlp_g(hh_g, agg_g, C_g, Ta[l], Tb[l], sa[l], sb[l], W1a[l], W1b[l],
                     b1p[l], W2a[l], W2b[l], b2a[l], b2b[l], gaa[l], gab[l],
                     baa[l], bab[l])
        hh_s = mlp_s(hh_s, agg_s, C_s, Ta[l], Tb[l], sa[l], sb[l], W1a[l], W1b[l],
                     b1p[l], W2a[l], W2b[l], b2a[l], b2b[l], gaa[l], gab[l],
                     baa[l], bab[l])

    S_g, c_g = pool_g(hh_g, P1a, P1b, pb1p, P2p, pb2p, seg_g, m_g)
    S_s, c_s = pool_s(hh_s, P1a, P1b, pb1p, P2p, pb2p, seg_s, m_s)
    logits = _final_logits(S_g, c_g, S_s, c_s)
    return (logits, graph_contrast_labels)
